# baseline (speedup 1.0000x reference)
import jax
import jax.numpy as jnp
from jax import lax
from jax.experimental import pallas as pl
from jax.experimental.pallas import tpu as pltpu
from jax.experimental.pallas import tpu_sc as plsc

VOCAB = 1000000
EMBED = 128
BATCH = 4096
NUM_DENSE = 13
NUM_CAT = 26

N_IDX = BATCH * NUM_CAT
NC, NS = 2, 16
NW = NC * NS
PER_W = N_IDX // NW
CHUNK = 128
N_CHUNK = PER_W // CHUNK
_NBUF = 4
_FPB = 2


def _gather_body(table_hbm, idx_hbm, out_hbm, idx_v, *rest):
    bufs = rest[:_NBUF]
    stages = rest[_NBUF:2 * _NBUF]
    gsems = rest[2 * _NBUF:3 * _NBUF]
    ssems = rest[3 * _NBUF:4 * _NBUF]
    wid = lax.axis_index("s") * NC + lax.axis_index("c")
    pltpu.sync_copy(idx_hbm.at[pl.ds(wid * PER_W, PER_W)], idx_v)

    lane = lax.iota(jnp.int32, 16) * NUM_CAT
    out_base = wid * CHUNK

    def build_stage(f, b):
        for i in range(CHUNK // 16):
            pos = lane + (NUM_CAT * 16 * i + f)
            stages[b][pl.ds(16 * i, 16)] = plsc.load_gather(idx_v, [pos])

    def gather(b):
        return pltpu.make_async_copy(table_hbm.at[stages[b]], bufs[b],
                                     gsems[b])

    def store(f, b):
        return pltpu.make_async_copy(
            bufs[b], out_hbm.at[pl.ds(f * BATCH + out_base, CHUNK)], ssems[b])

    for c in range(N_CHUNK + _NBUF - 1):
        if c < N_CHUNK:
            s = c % _NBUF
            if c >= _NBUF:
                store(c - _NBUF, s).wait()
            build_stage(c, s)
            gather(s).start()
        d = c - (_NBUF - 1)
        if 0 <= d < N_CHUNK:
            sd = d % _NBUF
            gather(sd).wait()
            store(d, sd).start()
    for d in range(max(0, N_CHUNK - _NBUF), N_CHUNK):
        store(d, d % _NBUF).wait()


def _sc_gather(table, idx_flat):
    mesh = plsc.VectorSubcoreMesh(core_axis_name="c", subcore_axis_name="s")
    return pl.kernel(
        _gather_body,
        out_type=jax.ShapeDtypeStruct((N_IDX, EMBED), jnp.float32),
        mesh=mesh,
        compiler_params=pltpu.CompilerParams(needs_layout_passes=False),
        scratch_types=(
            [pltpu.VMEM((PER_W,), jnp.int32)]
            + [pltpu.VMEM((CHUNK, EMBED), jnp.float32)] * _NBUF
            + [pltpu.VMEM((CHUNK,), jnp.int32)] * _NBUF
            + [pltpu.SemaphoreType.DMA] * (2 * _NBUF)
        ),
    )(table, idx_flat)


def _mlp_body(x_ref, emb_ref, wf_ref, wb0, bb0, wb1, bb1, wb2, bb2,
              wt0a, bt0, wt1, bt1, wt2, bt2, wt3, bt3, wo, bo,
              out_ref, acc_ref):
    f32 = jnp.float32
    bf16 = jnp.bfloat16
    f = pl.program_id(0)

    @pl.when(f == 0)
    def _init():
        xb = x_ref[:, :NUM_DENSE]
        bot = jax.nn.relu(jnp.dot(xb, wb0[:], preferred_element_type=f32) + bb0[:])
        bot = bot + jax.nn.relu(jnp.dot(bot, wb1[:], preferred_element_type=f32) + bb1[:])
        bot = bot + jax.nn.relu(jnp.dot(bot, wb2[:], preferred_element_type=f32) + bb2[:])
        acc_ref[:] = jnp.dot(bot, wt0a[:], preferred_element_type=f32) + bt0[:]

    wf = wf_ref[:].astype(bf16)
    acc_ref[:] += (
        jnp.dot(emb_ref[:BATCH].astype(bf16), wf[:EMBED],
                preferred_element_type=f32)
        + jnp.dot(emb_ref[BATCH:].astype(bf16), wf[EMBED:],
                  preferred_element_type=f32))

    @pl.when(f == NUM_CAT // _FPB - 1)
    def _finish():
        t = jax.nn.relu(acc_ref[:])
        t = t + jax.nn.relu(jnp.dot(t, wt1[:], preferred_element_type=f32) + bt1[:])
        t = t + jax.nn.relu(jnp.dot(t, wt2[:], preferred_element_type=f32) + bt2[:])
        t = t + jax.nn.relu(jnp.dot(t, wt3[:], preferred_element_type=f32) + bt3[:])
        out_ref[:] = jnp.dot(t, wo[:], preferred_element_type=f32) + bo[:]


def _tc_mlp(x, emb, wt0b, wb0, bb0, wb1, bb1, wb2, bb2,
            wt0a, bt0, wt1, bt1, wt2, bt2, wt3, bt3, wo, bo):
    def bspec(shape):
        return pl.BlockSpec(shape, lambda f: (0,) * len(shape))

    return pl.pallas_call(
        _mlp_body,
        grid=(NUM_CAT // _FPB,),
        in_specs=[
            pl.BlockSpec((BATCH, NUM_DENSE + NUM_CAT), lambda f: (0, 0)),
            pl.BlockSpec((_FPB * BATCH, EMBED), lambda f: (f, 0)),
            pl.BlockSpec((_FPB * EMBED, 256), lambda f: (f, 0)),
            bspec(wb0.shape), bspec(bb0.shape),
            bspec(wb1.shape), bspec(bb1.shape),
            bspec(wb2.shape), bspec(bb2.shape),
            bspec(wt0a.shape), bspec(bt0.shape),
            bspec(wt1.shape), bspec(bt1.shape),
            bspec(wt2.shape), bspec(bt2.shape),
            bspec(wt3.shape), bspec(bt3.shape),
            bspec(wo.shape), bspec(bo.shape),
        ],
        out_specs=pl.BlockSpec((BATCH, 1), lambda f: (0, 0)),
        out_shape=jax.ShapeDtypeStruct((BATCH, 1), jnp.float32),
        scratch_shapes=[pltpu.VMEM((BATCH, 256), jnp.float32)],
    )(x, emb, wt0b, wb0, bb0, wb1, bb1, wb2, bb2,
      wt0a, bt0, wt1, bt1, wt2, bt2, wt3, bt3, wo, bo)


def kernel(x, W_bot0, b_bot0, W_bot1, b_bot1, W_bot2, b_bot2, embedding_table,
           W_top0, b_top0, W_top1, b_top1, W_top2, b_top2, W_top3, b_top3,
           W_out, b_out):
    idx = jnp.asarray(x[:, NUM_DENSE:], jnp.int32) % VOCAB
    emb = _sc_gather(embedding_table, idx.reshape(-1))
    wt0a = W_top0[:256]
    wt0b = W_top0[256:]
    return _tc_mlp(
        x, emb, wt0b,
        W_bot0, b_bot0.reshape(1, -1),
        W_bot1, b_bot1.reshape(1, -1),
        W_bot2, b_bot2.reshape(1, -1),
        wt0a, b_top0.reshape(1, -1),
        W_top1, b_top1.reshape(1, -1),
        W_top2, b_top2.reshape(1, -1),
        W_top3, b_top3.reshape(1, -1),
        W_out, b_out.reshape(1, -1))


# speedup vs baseline: 1.0302x; 1.0302x over previous
import jax
import jax.numpy as jnp
from jax import lax
from jax.experimental import pallas as pl
from jax.experimental.pallas import tpu as pltpu
from jax.experimental.pallas import tpu_sc as plsc

VOCAB = 1000000
EMBED = 128
BATCH = 4096
NUM_DENSE = 13
NUM_CAT = 26

N_IDX = BATCH * NUM_CAT
NC, NS = 2, 16
NW = NC * NS
PER_W = N_IDX // NW
CHUNK = 128
N_CHUNK = PER_W // CHUNK
_NBUF = 4
_FPB = 2


def _gather_body(table_hbm, idx_hbm, out_hbm, idx_v, *rest):
    bufs = rest[:_NBUF]
    stages = rest[_NBUF:2 * _NBUF]
    gsems = rest[2 * _NBUF:3 * _NBUF]
    ssems = rest[3 * _NBUF:4 * _NBUF]
    wid = lax.axis_index("s") * NC + lax.axis_index("c")
    pltpu.sync_copy(idx_hbm.at[pl.ds(wid * PER_W, PER_W)], idx_v)

    lane = lax.iota(jnp.int32, 16) * NUM_CAT
    out_base = wid * CHUNK

    def build_stage(f, b):
        for i in range(CHUNK // 16):
            pos = lane + (NUM_CAT * 16 * i + f)
            stages[b][pl.ds(16 * i, 16)] = plsc.load_gather(idx_v, [pos])

    def gather(b):
        return pltpu.make_async_copy(table_hbm.at[stages[b]], bufs[b],
                                     gsems[b])

    def store(f, b):
        return pltpu.make_async_copy(
            bufs[b],
            out_hbm.at[pl.ds(out_base, CHUNK), pl.ds(f * EMBED, EMBED)],
            ssems[b])

    for c in range(N_CHUNK + _NBUF - 1):
        if c < N_CHUNK:
            s = c % _NBUF
            if c >= _NBUF:
                store(c - _NBUF, s).wait()
            build_stage(c, s)
            gather(s).start()
        d = c - (_NBUF - 1)
        if 0 <= d < N_CHUNK:
            sd = d % _NBUF
            gather(sd).wait()
            store(d, sd).start()
    for d in range(max(0, N_CHUNK - _NBUF), N_CHUNK):
        store(d, d % _NBUF).wait()


def _sc_gather(table, idx_flat):
    mesh = plsc.VectorSubcoreMesh(core_axis_name="c", subcore_axis_name="s")
    return pl.kernel(
        _gather_body,
        out_type=jax.ShapeDtypeStruct((BATCH, NUM_CAT * EMBED), jnp.float32),
        mesh=mesh,
        compiler_params=pltpu.CompilerParams(needs_layout_passes=False),
        scratch_types=(
            [pltpu.VMEM((PER_W,), jnp.int32)]
            + [pltpu.VMEM((CHUNK, EMBED), jnp.float32)] * _NBUF
            + [pltpu.VMEM((CHUNK,), jnp.int32)] * _NBUF
            + [pltpu.SemaphoreType.DMA] * (2 * _NBUF)
        ),
    )(table, idx_flat)


def _mlp_body(x_ref, emb_ref, wf_ref, wb0, bb0, wb1, bb1, wb2, bb2,
              wt0a, bt0, wt1, bt1, wt2, bt2, wt3, bt3, wo, bo,
              out_ref, acc_ref):
    f32 = jnp.float32
    f = pl.program_id(0)

    @pl.when(f == 0)
    def _init():
        xb = x_ref[:, :NUM_DENSE]
        bot = jax.nn.relu(jnp.dot(xb, wb0[:], preferred_element_type=f32) + bb0[:])
        bot = bot + jax.nn.relu(jnp.dot(bot, wb1[:], preferred_element_type=f32) + bb1[:])
        bot = bot + jax.nn.relu(jnp.dot(bot, wb2[:], preferred_element_type=f32) + bb2[:])
        acc_ref[:] = jnp.dot(bot, wt0a[:], preferred_element_type=f32) + bt0[:]

    acc_ref[:] += jnp.dot(emb_ref[:], wf_ref[:], preferred_element_type=f32)

    @pl.when(f == NUM_CAT // _FPB - 1)
    def _finish():
        t = jax.nn.relu(acc_ref[:])
        t = t + jax.nn.relu(jnp.dot(t, wt1[:], preferred_element_type=f32) + bt1[:])
        t = t + jax.nn.relu(jnp.dot(t, wt2[:], preferred_element_type=f32) + bt2[:])
        t = t + jax.nn.relu(jnp.dot(t, wt3[:], preferred_element_type=f32) + bt3[:])
        out_ref[:] = jnp.dot(t, wo[:], preferred_element_type=f32) + bo[:]


def _tc_mlp(x, emb, wt0b, wb0, bb0, wb1, bb1, wb2, bb2,
            wt0a, bt0, wt1, bt1, wt2, bt2, wt3, bt3, wo, bo):
    def bspec(shape):
        return pl.BlockSpec(shape, lambda f: (0,) * len(shape))

    return pl.pallas_call(
        _mlp_body,
        grid=(NUM_CAT // _FPB,),
        in_specs=[
            pl.BlockSpec((BATCH, NUM_DENSE + NUM_CAT), lambda f: (0, 0)),
            pl.BlockSpec((BATCH, _FPB * EMBED), lambda f: (0, f)),
            pl.BlockSpec((_FPB * EMBED, 256), lambda f: (f, 0)),
            bspec(wb0.shape), bspec(bb0.shape),
            bspec(wb1.shape), bspec(bb1.shape),
            bspec(wb2.shape), bspec(bb2.shape),
            bspec(wt0a.shape), bspec(bt0.shape),
            bspec(wt1.shape), bspec(bt1.shape),
            bspec(wt2.shape), bspec(bt2.shape),
            bspec(wt3.shape), bspec(bt3.shape),
            bspec(wo.shape), bspec(bo.shape),
        ],
        out_specs=pl.BlockSpec((BATCH, 1), lambda f: (0, 0)),
        out_shape=jax.ShapeDtypeStruct((BATCH, 1), jnp.float32),
        scratch_shapes=[pltpu.VMEM((BATCH, 256), jnp.float32)],
    )(x, emb, wt0b, wb0, bb0, wb1, bb1, wb2, bb2,
      wt0a, bt0, wt1, bt1, wt2, bt2, wt3, bt3, wo, bo)


def kernel(x, W_bot0, b_bot0, W_bot1, b_bot1, W_bot2, b_bot2, embedding_table,
           W_top0, b_top0, W_top1, b_top1, W_top2, b_top2, W_top3, b_top3,
           W_out, b_out):
    idx = jnp.asarray(x[:, NUM_DENSE:], jnp.int32) % VOCAB
    emb = _sc_gather(embedding_table, idx.reshape(-1))
    wt0a = W_top0[:256]
    wt0b = W_top0[256:]
    return _tc_mlp(
        x, emb, wt0b,
        W_bot0, b_bot0.reshape(1, -1),
        W_bot1, b_bot1.reshape(1, -1),
        W_bot2, b_bot2.reshape(1, -1),
        wt0a, b_top0.reshape(1, -1),
        W_top1, b_top1.reshape(1, -1),
        W_top2, b_top2.reshape(1, -1),
        W_top3, b_top3.reshape(1, -1),
        W_out, b_out.reshape(1, -1))


# speedup vs baseline: 1.0385x; 1.0081x over previous
import jax
import jax.numpy as jnp
from jax import lax
from jax.experimental import pallas as pl
from jax.experimental.pallas import tpu as pltpu
from jax.experimental.pallas import tpu_sc as plsc

VOCAB = 1000000
EMBED = 128
BATCH = 4096
NUM_DENSE = 13
NUM_CAT = 26

N_IDX = BATCH * NUM_CAT
NC, NS = 2, 16
NW = NC * NS
PER_W = N_IDX // NW
CHUNK = 128
N_CHUNK = PER_W // CHUNK
_NBUF = 4
_FPB = 2


def _gather_body(table_hbm, idx_hbm, out_hbm, idx_v, *rest):
    bufs = rest[:_NBUF]
    stages = rest[_NBUF:2 * _NBUF]
    gsems = rest[2 * _NBUF:3 * _NBUF]
    ssems = rest[3 * _NBUF:4 * _NBUF]
    wid = lax.axis_index("s") * NC + lax.axis_index("c")
    pltpu.sync_copy(idx_hbm.at[pl.ds(wid * PER_W, PER_W)], idx_v)

    lane = lax.iota(jnp.int32, 16) * NUM_CAT
    out_base = wid * CHUNK

    def build_stage(f, b):
        for i in range(CHUNK // 16):
            pos = lane + (NUM_CAT * 16 * i + f)
            stages[b][pl.ds(16 * i, 16)] = plsc.load_gather(idx_v, [pos])

    def gather(b):
        return pltpu.make_async_copy(table_hbm.at[stages[b]], bufs[b],
                                     gsems[b])

    def store(f, b):
        return pltpu.make_async_copy(
            bufs[b],
            out_hbm.at[f // _FPB, pl.ds(out_base, CHUNK),
                       pl.ds((f % _FPB) * EMBED, EMBED)],
            ssems[b])

    for c in range(N_CHUNK + _NBUF - 1):
        if c < N_CHUNK:
            s = c % _NBUF
            if c >= _NBUF:
                store(c - _NBUF, s).wait()
            build_stage(c, s)
            gather(s).start()
        d = c - (_NBUF - 1)
        if 0 <= d < N_CHUNK:
            sd = d % _NBUF
            gather(sd).wait()
            store(d, sd).start()
    for d in range(max(0, N_CHUNK - _NBUF), N_CHUNK):
        store(d, d % _NBUF).wait()


def _sc_gather(table, idx_flat):
    mesh = plsc.VectorSubcoreMesh(core_axis_name="c", subcore_axis_name="s")
    return pl.kernel(
        _gather_body,
        out_type=jax.ShapeDtypeStruct(
            (NUM_CAT // _FPB, BATCH, _FPB * EMBED), jnp.float32),
        mesh=mesh,
        compiler_params=pltpu.CompilerParams(needs_layout_passes=False),
        scratch_types=(
            [pltpu.VMEM((PER_W,), jnp.int32)]
            + [pltpu.VMEM((CHUNK, EMBED), jnp.float32)] * _NBUF
            + [pltpu.VMEM((CHUNK,), jnp.int32)] * _NBUF
            + [pltpu.SemaphoreType.DMA] * (2 * _NBUF)
        ),
    )(table, idx_flat)


def _mlp_body(x_ref, emb_ref, wf_ref, wb0, bb0, wb1, bb1, wb2, bb2,
              wt0a, bt0, wt1, bt1, wt2, bt2, wt3, bt3, wo, bo,
              out_ref, acc_ref):
    f32 = jnp.float32
    f = pl.program_id(0)

    @pl.when(f == 0)
    def _init():
        xb = x_ref[:, :NUM_DENSE]
        bot = jax.nn.relu(jnp.dot(xb, wb0[:], preferred_element_type=f32) + bb0[:])
        bot = bot + jax.nn.relu(jnp.dot(bot, wb1[:], preferred_element_type=f32) + bb1[:])
        bot = bot + jax.nn.relu(jnp.dot(bot, wb2[:], preferred_element_type=f32) + bb2[:])
        acc_ref[:] = jnp.dot(bot, wt0a[:], preferred_element_type=f32) + bt0[:]

    acc_ref[:] += jnp.dot(emb_ref[0], wf_ref[:], preferred_element_type=f32)

    @pl.when(f == NUM_CAT // _FPB - 1)
    def _finish():
        t = jax.nn.relu(acc_ref[:])
        t = t + jax.nn.relu(jnp.dot(t, wt1[:], preferred_element_type=f32) + bt1[:])
        t = t + jax.nn.relu(jnp.dot(t, wt2[:], preferred_element_type=f32) + bt2[:])
        t = t + jax.nn.relu(jnp.dot(t, wt3[:], preferred_element_type=f32) + bt3[:])
        out_ref[:] = jnp.dot(t, wo[:], preferred_element_type=f32) + bo[:]


def _tc_mlp(x, emb, wt0b, wb0, bb0, wb1, bb1, wb2, bb2,
            wt0a, bt0, wt1, bt1, wt2, bt2, wt3, bt3, wo, bo):
    def bspec(shape):
        return pl.BlockSpec(shape, lambda f: (0,) * len(shape))

    return pl.pallas_call(
        _mlp_body,
        grid=(NUM_CAT // _FPB,),
        in_specs=[
            pl.BlockSpec((BATCH, NUM_DENSE + NUM_CAT), lambda f: (0, 0)),
            pl.BlockSpec((1, BATCH, _FPB * EMBED), lambda f: (f, 0, 0)),
            pl.BlockSpec((_FPB * EMBED, 256), lambda f: (f, 0)),
            bspec(wb0.shape), bspec(bb0.shape),
            bspec(wb1.shape), bspec(bb1.shape),
            bspec(wb2.shape), bspec(bb2.shape),
            bspec(wt0a.shape), bspec(bt0.shape),
            bspec(wt1.shape), bspec(bt1.shape),
            bspec(wt2.shape), bspec(bt2.shape),
            bspec(wt3.shape), bspec(bt3.shape),
            bspec(wo.shape), bspec(bo.shape),
        ],
        out_specs=pl.BlockSpec((BATCH, 1), lambda f: (0, 0)),
        out_shape=jax.ShapeDtypeStruct((BATCH, 1), jnp.float32),
        scratch_shapes=[pltpu.VMEM((BATCH, 256), jnp.float32)],
    )(x, emb, wt0b, wb0, bb0, wb1, bb1, wb2, bb2,
      wt0a, bt0, wt1, bt1, wt2, bt2, wt3, bt3, wo, bo)


def kernel(x, W_bot0, b_bot0, W_bot1, b_bot1, W_bot2, b_bot2, embedding_table,
           W_top0, b_top0, W_top1, b_top1, W_top2, b_top2, W_top3, b_top3,
           W_out, b_out):
    idx = jnp.asarray(x[:, NUM_DENSE:], jnp.int32) % VOCAB
    emb = _sc_gather(embedding_table, idx.reshape(-1))
    wt0a = W_top0[:256]
    wt0b = W_top0[256:]
    return _tc_mlp(
        x, emb, wt0b,
        W_bot0, b_bot0.reshape(1, -1),
        W_bot1, b_bot1.reshape(1, -1),
        W_bot2, b_bot2.reshape(1, -1),
        wt0a, b_top0.reshape(1, -1),
        W_top1, b_top1.reshape(1, -1),
        W_top2, b_top2.reshape(1, -1),
        W_top3, b_top3.reshape(1, -1),
        W_out, b_out.reshape(1, -1))
